# baseline (speedup 1.0000x reference)
import functools

import jax
import jax.numpy as jnp
from jax import lax
from jax.experimental import pallas as pl
from jax.experimental.pallas import tpu as pltpu
from jax.experimental.pallas import tpu_sc as plsc

R = 2.0
T = 1.0

NC = 2
NS = 16
NW = NC * NS
L = 16

N_NODES = 10000
D = 128
N_EDGES = 320000
E_W = N_EDGES // NW
CHUNK = 80
NCH = E_W // CHUNK
NBUF = 4


def _renorm_tc(h):
    blk = 1000

    def body(h_ref, o_ref):
        x = h_ref[...]
        ss = jnp.sum(x * x, axis=1, keepdims=True)
        norm = jnp.sqrt(ss)
        scale = jnp.where(norm > 1.0, 1.0 / jnp.maximum(norm, 1e-12), 1.0)
        o_ref[...] = x * scale

    return pl.pallas_call(
        body,
        out_shape=jax.ShapeDtypeStruct((N_NODES, D), jnp.float32),
        grid=(N_NODES // blk,),
        in_specs=[pl.BlockSpec((blk, D), lambda i: (i, 0))],
        out_specs=pl.BlockSpec((blk, D), lambda i: (i, 0)),
    )(h)


def _decode_sc(tab, idx0, idx1):
    mesh = plsc.VectorSubcoreMesh(core_axis_name="c", subcore_axis_name="s")

    @functools.partial(
        pl.kernel,
        out_type=jax.ShapeDtypeStruct((N_EDGES,), jnp.float32),
        mesh=mesh,
        scratch_types=[
            pltpu.VMEM((E_W,), jnp.int32),
            pltpu.VMEM((E_W,), jnp.int32),
            pltpu.VMEM((NBUF, CHUNK, D), jnp.float32),
            pltpu.VMEM((NBUF, CHUNK, D), jnp.float32),
            pltpu.VMEM((E_W,), jnp.float32),
            pltpu.SemaphoreType.DMA,
            pltpu.SemaphoreType.DMA,
            pltpu.SemaphoreType.DMA,
            pltpu.SemaphoreType.DMA,
        ],
    )
    def decode(tab_hbm, idx0_hbm, idx1_hbm, out_hbm,
               idx0_all, idx1_all, rows0, rows1, out_all,
               sem0, sem1, sem2, sem3):
        sems = [sem0, sem1, sem2, sem3]
        wid = lax.axis_index("s") * NC + lax.axis_index("c")
        base = wid * E_W

        pltpu.sync_copy(idx0_hbm.at[pl.ds(base, E_W)], idx0_all)
        pltpu.sync_copy(idx1_hbm.at[pl.ds(base, E_W)], idx1_all)

        def start(ci, b):
            off = ci * CHUNK
            pltpu.async_copy(tab_hbm.at[idx0_all.at[pl.ds(off, CHUNK)]],
                             rows0.at[b], sems[b])
            pltpu.async_copy(tab_hbm.at[idx1_all.at[pl.ds(off, CHUNK)]],
                             rows1.at[b], sems[b])

        def wait(b):
            pltpu.make_async_copy(tab_hbm.at[pl.ds(0, CHUNK)],
                                  rows0.at[b], sems[b]).wait()
            pltpu.make_async_copy(tab_hbm.at[pl.ds(0, CHUNK)],
                                  rows1.at[b], sems[b]).wait()

        lane = lax.iota(jnp.int32, L)
        perms = [(lane ^ (1 << k))[:, None] for k in range(4)]
        dnums = lax.GatherDimensionNumbers(
            offset_dims=(), collapsed_slice_dims=(0,),
            start_index_map=(0,))

        def lane_sum(v):
            for p in perms:
                v = v + lax.gather(
                    v, p, dnums, slice_sizes=(1,),
                    mode=lax.GatherScatterMode.PROMISE_IN_BOUNDS)
            return v

        def compute(ci, b):
            obase = ci * CHUNK
            r0 = rows0.at[b]
            r1 = rows1.at[b]

            def group_body(g, c2):
                res = jnp.zeros((L,), jnp.float32)
                for k in range(L):
                    e = g * L + k
                    acc = jnp.zeros((L,), jnp.float32)
                    for d in range(D // L):
                        a = r0[e, pl.ds(d * L, L)]
                        b = r1[e, pl.ds(d * L, L)]
                        df = a - b
                        acc = acc + df * df
                    res = jnp.where(lane == k, lane_sum(acc), res)
                out_all[pl.ds(obase + g * L, L)] = (
                    1.0 / (jnp.exp((res - R) / T) + 1.0))
                return c2

            lax.fori_loop(0, CHUNK // L, group_body, 0, unroll=False)

        for b in range(NBUF):
            start(b, b)

        def ring_body(gg, carry):
            for b in range(NBUF):
                ci = NBUF * gg + b
                wait(b)
                compute(ci, b)

                @pl.when(ci + NBUF < NCH)
                def _():
                    start(ci + NBUF, b)

            return carry

        lax.fori_loop(0, NCH // NBUF, ring_body, 0, unroll=False)

        for ci in range(NCH - NCH % NBUF, NCH):
            b = ci % NBUF
            wait(b)
            compute(ci, b)

        pltpu.sync_copy(out_all, out_hbm.at[pl.ds(base, E_W)])

    return decode(tab, idx0, idx1)


def kernel(h, idx):
    idx = idx.astype(jnp.int32)
    idx0 = idx[:, 0]
    idx1 = idx[:, 1]
    tab = _renorm_tc(h)
    return _decode_sc(tab, idx0, idx1)
